# baseline (speedup 1.0000x reference)
import math

import numpy as np
import jax
import jax.numpy as jnp
from jax import lax
from jax.experimental import pallas as pl
from jax.experimental.pallas import tpu as pltpu
from jax.experimental.pallas import tpu_sc as plsc

_IN = (128, 256)
_OUT = (64, 128)
_K = 3
_C = 64
_CUT = _K * math.pi / _IN[0]

NW = 32
PL = 32
ACCW = PL * _C
WLAT = 6
HROW = (128 + PL) * _C
WINW = 2 * WLAT * HROW


def _nz_pattern():
    nlat_in, nlon_in = _IN
    nlat_out = _OUT[0]
    lats_in = np.linspace(0.0, np.pi, nlat_in)
    lats_out = np.linspace(0.0, np.pi, nlat_out)
    phis = np.linspace(0.0, 2.0 * np.pi, nlon_in)
    d_theta = _CUT / _K
    alpha = -lats_in.reshape(-1, 1)
    beta = phis.reshape(1, -1)
    nz = {}
    for t in range(nlat_out):
        gamma = lats_out[t]
        z = np.cos(alpha) * np.cos(gamma) - np.cos(beta) * np.sin(alpha) * np.sin(gamma)
        theta = np.arccos(np.clip(z, -1.0, 1.0))
        for ik in range(_K):
            diff = np.abs(theta - ik * d_theta)
            mask = (diff < d_theta) & (theta < _CUT)
            ii, jj = np.nonzero(mask)
            nz[(ik, t)] = (ii.astype(np.int64), jj.astype(np.int64))
    return nz


_ZERO_PSI_IDX = 32 * _IN[0] * _IN[1]


def _build_schedule():
    nlat_out = _OUT[0]
    nz = _nz_pattern()
    assert 0 not in (nz[(0, 32)][0] * _IN[1] + nz[(0, 32)][1]).tolist()

    ilo_t = np.zeros(nlat_out, dtype=np.int64)
    for t in range(nlat_out):
        allio = np.concatenate([nz[(k, t)][0] for k in range(_K)])
        lo, hi = int(allio.min()), int(allio.max())
        assert hi - lo + 1 <= WLAT
        ilo_t[t] = min(max(lo, 0), _IN[0] - WLAT)

    nchunks = 128 // PL
    items_by_t = {t: [] for t in range(nlat_out)}
    cost_t = np.zeros(nlat_out)
    for t in range(nlat_out):
        for k in range(_K):
            cnt = len(nz[(k, t)][0])
            for ch in range(nchunks):
                items_by_t[t].append((k, ch, cnt * PL))
            cost_t[t] += cnt * 128

    PEN = 8000.0
    load = np.zeros(NW)
    bin_ts = [set() for _ in range(NW)]
    bin_items = [[] for _ in range(NW)]
    for t in sorted(range(nlat_out), key=lambda t: -cost_t[t]):
        for (k, ch, cost) in sorted(items_by_t[t], key=lambda it: -it[2]):
            best, bestv = 0, None
            for w in range(NW):
                v = load[w] + (0.0 if t in bin_ts[w] else PEN)
                if bestv is None or v < bestv:
                    best, bestv = w, v
            load[best] += cost
            bin_ts[best].add(t)
            bin_items[best].append((t, k, ch))

    hdr, g_ilo, g_nit = [], [], []
    it_outb, it_nzc4 = [], []
    nz_w_idx, nz_qw_i, nz_s1 = [], [], []
    per_bin_counts = []
    for w in range(NW):
        groups = {}
        for (t, k, ch) in bin_items[w]:
            groups.setdefault(t, []).append((k, ch))
        hdr.append(len(groups))
        bg_ilo, bg_nit = [], []
        bi_outb, bi_nzc4 = [], []
        bn = 0
        for t, its in groups.items():
            bg_ilo.append(int(ilo_t[t]))
            bg_nit.append(len(its))
            for (k, ch) in its:
                ii, jj = nz[(k, t)]
                p0 = ch * PL
                r = k * nlat_out + t
                bi_outb.append(r * 128 * _C + p0 * _C)
                par = jj % 2
                m0 = jj // 2
                s = (128 - m0) % 128
                vstart = (p0 + s) % 128
                di = ii - ilo_t[t]
                rb = (par * WLAT + di) * HROW
                s1 = (rb + vstart * _C).tolist()
                widx = (k * (nlat_out * _IN[0] * _IN[1])
                        + t * (_IN[0] * _IN[1]) + ii * _IN[1] + jj).tolist()
                qwi = ii.tolist()
                npad = (-len(s1)) % 4
                s1 += [0] * npad
                widx += [_ZERO_PSI_IDX] * npad
                qwi += [0] * npad
                bi_nzc4.append(len(s1) // 4)
                nz_s1.extend(s1)
                nz_w_idx.extend(widx)
                nz_qw_i.extend(qwi)
                bn += len(s1)
        g_ilo.append(bg_ilo)
        g_nit.append(bg_nit)
        it_outb.append(bi_outb)
        it_nzc4.append(bi_nzc4)
        per_bin_counts.append((len(bg_ilo), len(bi_outb), bn))

    def _pad16(v):
        return (v + 16 + 15) // 16 * 16

    maxg = _pad16(max(c[0] for c in per_bin_counts))
    maxm = _pad16(max(c[1] for c in per_bin_counts))
    maxn = max(c[2] for c in per_bin_counts) + 16
    maxn = (maxn + 127) // 128 * 128

    A_hdr = np.zeros((NW, 16), dtype=np.int32)
    A_gilo = np.zeros((NW, maxg), dtype=np.int32)
    A_gnit = np.zeros((NW, maxg), dtype=np.int32)
    A_outb = np.zeros((NW, maxm), dtype=np.int32)
    A_nzc4 = np.zeros((NW, maxm), dtype=np.int32)
    A_s1 = np.zeros((NW, maxn), dtype=np.int32)
    A_widx = np.full((NW, maxn), _ZERO_PSI_IDX, dtype=np.int32)
    A_qwi = np.zeros((NW, maxn), dtype=np.int32)
    off = 0
    for w in range(NW):
        ng, nm, bn = per_bin_counts[w]
        A_hdr[w, 0] = ng
        A_gilo[w, :ng] = g_ilo[w]
        A_gnit[w, :ng] = g_nit[w]
        A_outb[w, :nm] = it_outb[w]
        A_nzc4[w, :nm] = it_nzc4[w]
        A_s1[w, :bn] = nz_s1[off:off + bn]
        A_widx[w, :bn] = nz_w_idx[off:off + bn]
        A_qwi[w, :bn] = nz_qw_i[off:off + bn]
        off += bn
    return (A_hdr, A_gilo, A_gnit, A_outb, A_nzc4, A_s1, A_widx, A_qwi, maxn)


_SCHED = _build_schedule()
_MAXN = _SCHED[8]
_NGCH = _MAXN // 128


def _sc_body(xrev_ref, psi_ref, qw_ref, widx_ref, qwi_ref, hdr_ref, gilo_ref,
             gnit_ref, outb_ref, nzc_ref, s1_ref, out_ref,
             win_v, acc_v, wv_v, idx_v, qw_v, hdr_v, gilo_v, gnit_v,
             outb_v, nzc_v, s1_v, sem):
    wid = lax.axis_index("s") * 2 + lax.axis_index("c")

    pltpu.sync_copy(hdr_ref.at[wid], hdr_v)
    pltpu.sync_copy(gilo_ref.at[wid], gilo_v)
    pltpu.sync_copy(gnit_ref.at[wid], gnit_v)
    pltpu.sync_copy(outb_ref.at[wid], outb_v)
    pltpu.sync_copy(nzc_ref.at[wid], nzc_v)
    pltpu.sync_copy(s1_ref.at[wid], s1_v)
    pltpu.sync_copy(widx_ref.at[wid], idx_v)
    pltpu.sync_copy(qw_ref, qw_v)

    for c in range(_NGCH):
        pltpu.async_copy(psi_ref.at[idx_v.at[pl.ds(c * 128, 128)]],
                         wv_v.at[pl.ds(c * 128, 128)], sem).wait()

    pltpu.sync_copy(qwi_ref.at[wid], idx_v)

    @plsc.parallel_loop(0, _MAXN, 16, unroll=2)
    def qw_scale(o):
        qv = plsc.load_gather(qw_v, [idx_v[pl.ds(o, 16)]])
        wv_v[pl.ds(o, 16)] = wv_v[pl.ds(o, 16)] * qv

    def _sld(ref, i):
        return ref[pl.ds(i, 16)][0]

    ng = _sld(hdr_v, 0)
    zero16 = jnp.zeros((16,), jnp.float32)

    def group_body(g, carry):
        item_ptr, nz_ptr = carry
        ilo = _sld(gilo_v, g)
        base0 = pl.multiple_of(ilo * HROW, HROW)
        base1 = pl.multiple_of((128 + ilo) * HROW, HROW)
        pltpu.sync_copy(xrev_ref.at[pl.ds(base0, WLAT * HROW)],
                        win_v.at[pl.ds(0, WLAT * HROW)])
        pltpu.sync_copy(xrev_ref.at[pl.ds(base1, WLAT * HROW)],
                        win_v.at[pl.ds(WLAT * HROW, WLAT * HROW)])
        nitems = _sld(gnit_v, g)

        def item_body(it, carry2):
            ip, nzp = carry2
            outb = pl.multiple_of(_sld(outb_v, ip), ACCW)
            nzc4 = _sld(nzc_v, ip)

            @plsc.parallel_loop(0, ACCW, 16, unroll=4)
            def zero_body(o):
                acc_v[pl.ds(o, 16)] = zero16

            def nz_body(n, _):
                base = nzp + n * 4
                w0 = _sld(wv_v, base)
                w1 = _sld(wv_v, base + 1)
                w2 = _sld(wv_v, base + 2)
                w3 = _sld(wv_v, base + 3)
                e0 = pl.multiple_of(_sld(s1_v, base), 16)
                e1 = pl.multiple_of(_sld(s1_v, base + 1), 16)
                e2 = pl.multiple_of(_sld(s1_v, base + 2), 16)
                e3 = pl.multiple_of(_sld(s1_v, base + 3), 16)

                @plsc.parallel_loop(0, ACCW, 16, unroll=2)
                def fma_body(o):
                    v = acc_v[pl.ds(o, 16)]
                    v = v + (w0 * win_v[pl.ds(e0 + o, 16)]
                             + w1 * win_v[pl.ds(e1 + o, 16)])
                    v = v + (w2 * win_v[pl.ds(e2 + o, 16)]
                             + w3 * win_v[pl.ds(e3 + o, 16)])
                    acc_v[pl.ds(o, 16)] = v
                return 0

            lax.fori_loop(0, nzc4, nz_body, 0)
            pltpu.sync_copy(acc_v, out_ref.at[pl.ds(outb, ACCW)])
            return (ip + 1, nzp + (nzc4 << 2))

        return lax.fori_loop(0, nitems, item_body, (item_ptr, nz_ptr))

    lax.fori_loop(0, ng, group_body, (jnp.int32(0), jnp.int32(0)))


def kernel(x, psi, quad_weights):
    (A_hdr, A_gilo, A_gnit, A_outb, A_nzc4, A_s1, A_widx, A_qwi, maxn) = _SCHED

    qw = quad_weights.reshape(-1).astype(jnp.float32)
    psi_flat = psi.reshape(-1)

    xs = x.reshape(_C, _IN[0], _IN[1])
    xt = jnp.transpose(xs, (1, 2, 0))
    xp = xt.reshape(_IN[0], 128, 2, _C)
    xp = jnp.transpose(xp, (2, 0, 1, 3))
    xr4 = jnp.flip(xp, axis=2)
    xrev = jnp.concatenate([xr4, xr4[:, :, :PL, :]], axis=2).reshape(-1)

    mesh = plsc.VectorSubcoreMesh(core_axis_name="c", subcore_axis_name="s",
                                  num_cores=2, num_subcores=16)
    maxg = A_gilo.shape[1]
    maxm = A_outb.shape[1]
    out_flat = pl.kernel(
        _sc_body,
        out_type=jax.ShapeDtypeStruct((_K * _OUT[0] * 128 * _C,), jnp.float32),
        mesh=mesh,
        compiler_params=pltpu.CompilerParams(needs_layout_passes=False),
        scratch_types=[
            pltpu.VMEM((WINW,), jnp.float32),
            pltpu.VMEM((ACCW,), jnp.float32),
            pltpu.VMEM((maxn,), jnp.float32),
            pltpu.VMEM((maxn,), jnp.int32),
            pltpu.VMEM((128,), jnp.float32),
            pltpu.VMEM((16,), jnp.int32),
            pltpu.VMEM((maxg,), jnp.int32),
            pltpu.VMEM((maxg,), jnp.int32),
            pltpu.VMEM((maxm,), jnp.int32),
            pltpu.VMEM((maxm,), jnp.int32),
            pltpu.VMEM((maxn,), jnp.int32),
            pltpu.SemaphoreType.DMA,
        ],
    )(xrev, psi_flat, qw,
      jnp.asarray(A_widx), jnp.asarray(A_qwi),
      jnp.asarray(A_hdr), jnp.asarray(A_gilo), jnp.asarray(A_gnit),
      jnp.asarray(A_outb), jnp.asarray(A_nzc4), jnp.asarray(A_s1))

    out = out_flat.reshape(_K, _OUT[0], 128, _C)
    out = jnp.transpose(out, (3, 0, 1, 2))
    return out.reshape(1, _C, _K, _OUT[0], _OUT[1])


# speedup vs baseline: 1.1004x; 1.1004x over previous
import math

import numpy as np
import jax
import jax.numpy as jnp
from jax import lax
from jax.experimental import pallas as pl
from jax.experimental.pallas import tpu as pltpu
from jax.experimental.pallas import tpu_sc as plsc

_IN = (128, 256)
_OUT = (64, 128)
_K = 3
_C = 64
_CUT = _K * math.pi / _IN[0]

NW = 32
PL = 32
ACCW = PL * _C
WLAT = 6
HROW = (128 + PL) * _C
WINW = 2 * WLAT * HROW


def _nz_pattern():
    nlat_in, nlon_in = _IN
    nlat_out = _OUT[0]
    lats_in = np.linspace(0.0, np.pi, nlat_in)
    lats_out = np.linspace(0.0, np.pi, nlat_out)
    phis = np.linspace(0.0, 2.0 * np.pi, nlon_in)
    d_theta = _CUT / _K
    alpha = -lats_in.reshape(-1, 1)
    beta = phis.reshape(1, -1)
    nz = {}
    for t in range(nlat_out):
        gamma = lats_out[t]
        z = np.cos(alpha) * np.cos(gamma) - np.cos(beta) * np.sin(alpha) * np.sin(gamma)
        theta = np.arccos(np.clip(z, -1.0, 1.0))
        for ik in range(_K):
            diff = np.abs(theta - ik * d_theta)
            mask = (diff < d_theta) & (theta < _CUT)
            ii, jj = np.nonzero(mask)
            nz[(ik, t)] = (ii.astype(np.int64), jj.astype(np.int64))
    return nz


_ZERO_PSI_IDX = 32 * _IN[0] * _IN[1]


def _build_schedule():
    nlat_out = _OUT[0]
    nz = _nz_pattern()
    assert 0 not in (nz[(0, 32)][0] * _IN[1] + nz[(0, 32)][1]).tolist()

    ilo_t = np.zeros(nlat_out, dtype=np.int64)
    for t in range(nlat_out):
        allio = np.concatenate([nz[(k, t)][0] for k in range(_K)])
        lo, hi = int(allio.min()), int(allio.max())
        assert hi - lo + 1 <= WLAT
        ilo_t[t] = min(max(lo, 0), _IN[0] - WLAT)

    row_set = {}
    for t in range(nlat_out):
        for k in range(_K):
            ii, jj = nz[(k, t)]
            row_set[(k, t)] = set(zip((jj % 2).tolist(), ii.tolist()))

    nchunks = 128 // PL
    items_by_t = {t: [] for t in range(nlat_out)}
    cost_t = np.zeros(nlat_out)
    for t in range(nlat_out):
        for k in range(_K):
            cnt = len(nz[(k, t)][0])
            for ch in range(nchunks):
                items_by_t[t].append((k, ch, cnt * PL))
            cost_t[t] += cnt * 128

    ROWPEN = 650.0
    load = np.zeros(NW)
    bin_rows = [set() for _ in range(NW)]
    bin_items = [[] for _ in range(NW)]
    for t in sorted(range(nlat_out), key=lambda t: -cost_t[t]):
        for (k, ch, cost) in sorted(items_by_t[t], key=lambda it: -it[2]):
            rows = row_set[(k, t)]
            best, bestv = 0, None
            for w in range(NW):
                v = load[w] + ROWPEN * len(rows - bin_rows[w])
                if bestv is None or v < bestv:
                    best, bestv = w, v
            load[best] += cost
            bin_rows[best] |= rows
            bin_items[best].append((t, k, ch))

    hdr, g_ncp, g_nit = [], [], []
    cs_dst, cs_src = [], []
    it_outb, it_nzc4 = [], []
    nz_w_idx, nz_qw_i, nz_s1 = [], [], []
    per_bin_counts = []
    for w in range(NW):
        groups = {}
        for (t, k, ch) in bin_items[w]:
            groups.setdefault(t, []).append((k, ch))
        hdr.append(len(groups))
        bg_ncp, bg_nit = [], []
        bi_outb, bi_nzc4 = [], []
        bn = 0
        slot_row = [None] * (2 * WLAT)
        slot_age = [0] * (2 * WLAT)
        clock = 0
        bcs_dst, bcs_src = [], []
        for t in sorted(groups, key=lambda t: int(ilo_t[t])):
            its = groups[t]
            need = sorted(set().union(*[row_set[(k, t)] for (k, ch) in its]))
            clock += 1
            ncp = 0
            rowslot = {}
            for r in need:
                if r in slot_row:
                    sl = slot_row.index(r)
                    slot_age[sl] = clock
                    rowslot[r] = sl
            for r in need:
                if r in rowslot:
                    continue
                sl = min((s for s in range(2 * WLAT)
                          if slot_age[s] < clock),
                         key=lambda s: slot_age[s])
                slot_row[sl] = r
                slot_age[sl] = clock
                rowslot[r] = sl
                bcs_dst.append(sl * HROW)
                bcs_src.append((r[0] * 128 + r[1]) * HROW)
                ncp += 1
            bg_ncp.append(ncp)
            bg_nit.append(len(its))
            for (k, ch) in its:
                ii, jj = nz[(k, t)]
                p0 = ch * PL
                r = k * nlat_out + t
                bi_outb.append(r * 128 * _C + p0 * _C)
                par = jj % 2
                m0 = jj // 2
                s = (128 - m0) % 128
                vstart = (p0 + s) % 128
                rb = np.array([rowslot[(int(par[q]), int(ii[q]))] * HROW
                               for q in range(len(ii))], dtype=np.int64)
                s1 = (rb + vstart * _C).tolist()
                widx = (k * (nlat_out * _IN[0] * _IN[1])
                        + t * (_IN[0] * _IN[1]) + ii * _IN[1] + jj).tolist()
                qwi = ii.tolist()
                npad = (-len(s1)) % 4
                s1 += [0] * npad
                widx += [_ZERO_PSI_IDX] * npad
                qwi += [0] * npad
                bi_nzc4.append(len(s1) // 4)
                nz_s1.extend(s1)
                nz_w_idx.extend(widx)
                nz_qw_i.extend(qwi)
                bn += len(s1)
        g_ncp.append(bg_ncp)
        g_nit.append(bg_nit)
        cs_dst.append(bcs_dst)
        cs_src.append(bcs_src)
        it_outb.append(bi_outb)
        it_nzc4.append(bi_nzc4)
        per_bin_counts.append((len(bg_ncp), len(bi_outb), bn, len(bcs_dst)))

    def _pad16(v):
        return (v + 16 + 15) // 16 * 16

    maxg = _pad16(max(c[0] for c in per_bin_counts))
    maxm = _pad16(max(c[1] for c in per_bin_counts))
    maxc = _pad16(max(c[3] for c in per_bin_counts))
    maxn = max(c[2] for c in per_bin_counts) + 16
    maxn = (maxn + 127) // 128 * 128

    A_hdr = np.zeros((NW, 16), dtype=np.int32)
    A_gncp = np.zeros((NW, maxg), dtype=np.int32)
    A_gnit = np.zeros((NW, maxg), dtype=np.int32)
    A_csd = np.zeros((NW, maxc), dtype=np.int32)
    A_css = np.zeros((NW, maxc), dtype=np.int32)
    A_outb = np.zeros((NW, maxm), dtype=np.int32)
    A_nzc4 = np.zeros((NW, maxm), dtype=np.int32)
    A_s1 = np.zeros((NW, maxn), dtype=np.int32)
    A_widx = np.full((NW, maxn), _ZERO_PSI_IDX, dtype=np.int32)
    A_qwi = np.zeros((NW, maxn), dtype=np.int32)
    off = 0
    for w in range(NW):
        ng, nm, bn, nc = per_bin_counts[w]
        A_hdr[w, 0] = ng
        A_gncp[w, :ng] = g_ncp[w]
        A_gnit[w, :ng] = g_nit[w]
        A_csd[w, :nc] = cs_dst[w]
        A_css[w, :nc] = cs_src[w]
        A_outb[w, :nm] = it_outb[w]
        A_nzc4[w, :nm] = it_nzc4[w]
        A_s1[w, :bn] = nz_s1[off:off + bn]
        A_widx[w, :bn] = nz_w_idx[off:off + bn]
        A_qwi[w, :bn] = nz_qw_i[off:off + bn]
        off += bn
    return (A_hdr, A_gncp, A_gnit, A_csd, A_css, A_outb, A_nzc4, A_s1,
            A_widx, A_qwi, maxn)


_SCHED = _build_schedule()
_MAXN = _SCHED[10]
_NGCH = _MAXN // 128


def _sc_body(xrev_ref, psi_ref, qw_ref, widx_ref, qwi_ref, hdr_ref, gncp_ref,
             gnit_ref, csd_ref, css_ref, outb_ref, nzc_ref, s1_ref, out_ref,
             win_v, acc_v, wv_v, idx_v, qw_v, hdr_v, gncp_v, gnit_v,
             csd_v, css_v, outb_v, nzc_v, s1_v, sem):
    wid = lax.axis_index("s") * 2 + lax.axis_index("c")

    pltpu.sync_copy(hdr_ref.at[wid], hdr_v)
    pltpu.sync_copy(gncp_ref.at[wid], gncp_v)
    pltpu.sync_copy(gnit_ref.at[wid], gnit_v)
    pltpu.sync_copy(csd_ref.at[wid], csd_v)
    pltpu.sync_copy(css_ref.at[wid], css_v)
    pltpu.sync_copy(outb_ref.at[wid], outb_v)
    pltpu.sync_copy(nzc_ref.at[wid], nzc_v)
    pltpu.sync_copy(s1_ref.at[wid], s1_v)
    pltpu.sync_copy(widx_ref.at[wid], idx_v)
    pltpu.sync_copy(qw_ref, qw_v)

    for c in range(_NGCH):
        pltpu.async_copy(psi_ref.at[idx_v.at[pl.ds(c * 128, 128)]],
                         wv_v.at[pl.ds(c * 128, 128)], sem).wait()

    pltpu.sync_copy(qwi_ref.at[wid], idx_v)

    @plsc.parallel_loop(0, _MAXN, 16, unroll=2)
    def qw_scale(o):
        qv = plsc.load_gather(qw_v, [idx_v[pl.ds(o, 16)]])
        wv_v[pl.ds(o, 16)] = wv_v[pl.ds(o, 16)] * qv

    def _sld(ref, i):
        return ref[pl.ds(i, 16)][0]

    ng = _sld(hdr_v, 0)

    def group_body(g, carry):
        item_ptr, nz_ptr, cp_ptr = carry
        ncp = _sld(gncp_v, g)

        def fire(j, _):
            dst = pl.multiple_of(_sld(csd_v, cp_ptr + j), HROW)
            srw = pl.multiple_of(_sld(css_v, cp_ptr + j), HROW)
            pltpu.async_copy(xrev_ref.at[pl.ds(srw, HROW)],
                             win_v.at[pl.ds(dst, HROW)], sem)
            return 0
        lax.fori_loop(0, ncp, fire, 0)

        def drain(j, _):
            pltpu.make_async_copy(xrev_ref.at[pl.ds(0, HROW)],
                                  win_v.at[pl.ds(0, HROW)], sem).wait()
            return 0
        lax.fori_loop(0, ncp, drain, 0)
        nitems = _sld(gnit_v, g)

        def item_body(it, carry2):
            ip, nzp = carry2
            outb = pl.multiple_of(_sld(outb_v, ip), ACCW)
            nzc4 = _sld(nzc_v, ip)

            def _quad(base):
                w0 = _sld(wv_v, base)
                w1 = _sld(wv_v, base + 1)
                w2 = _sld(wv_v, base + 2)
                w3 = _sld(wv_v, base + 3)
                e0 = pl.multiple_of(_sld(s1_v, base), 16)
                e1 = pl.multiple_of(_sld(s1_v, base + 1), 16)
                e2 = pl.multiple_of(_sld(s1_v, base + 2), 16)
                e3 = pl.multiple_of(_sld(s1_v, base + 3), 16)
                return w0, w1, w2, w3, e0, e1, e2, e3

            w0, w1, w2, w3, e0, e1, e2, e3 = _quad(nzp)

            @plsc.parallel_loop(0, ACCW, 16, unroll=4)
            def init_body(o):
                acc_v[pl.ds(o, 16)] = (
                    (w0 * win_v[pl.ds(e0 + o, 16)]
                     + w1 * win_v[pl.ds(e1 + o, 16)])
                    + (w2 * win_v[pl.ds(e2 + o, 16)]
                       + w3 * win_v[pl.ds(e3 + o, 16)]))

            def nz_body(n, _):
                w0, w1, w2, w3, e0, e1, e2, e3 = _quad(nzp + n * 4)

                @plsc.parallel_loop(0, ACCW, 16, unroll=4)
                def fma_body(o):
                    v = acc_v[pl.ds(o, 16)]
                    v = v + (w0 * win_v[pl.ds(e0 + o, 16)]
                             + w1 * win_v[pl.ds(e1 + o, 16)])
                    v = v + (w2 * win_v[pl.ds(e2 + o, 16)]
                             + w3 * win_v[pl.ds(e3 + o, 16)])
                    acc_v[pl.ds(o, 16)] = v
                return 0

            lax.fori_loop(1, nzc4, nz_body, 0)
            pltpu.sync_copy(acc_v, out_ref.at[pl.ds(outb, ACCW)])
            return (ip + 1, nzp + (nzc4 << 2))

        ip2, nzp2 = lax.fori_loop(0, nitems, item_body, (item_ptr, nz_ptr))
        return (ip2, nzp2, cp_ptr + ncp)

    lax.fori_loop(0, ng, group_body,
                  (jnp.int32(0), jnp.int32(0), jnp.int32(0)))


def kernel(x, psi, quad_weights):
    (A_hdr, A_gncp, A_gnit, A_csd, A_css, A_outb, A_nzc4, A_s1,
     A_widx, A_qwi, maxn) = _SCHED

    qw = quad_weights.reshape(-1).astype(jnp.float32)
    psi_flat = psi.reshape(-1)

    xs = x.reshape(_C, _IN[0], _IN[1])
    xt = jnp.transpose(xs, (1, 2, 0))
    xp = xt.reshape(_IN[0], 128, 2, _C)
    xp = jnp.transpose(xp, (2, 0, 1, 3))
    xr4 = jnp.flip(xp, axis=2)
    xrev = jnp.concatenate([xr4, xr4[:, :, :PL, :]], axis=2).reshape(-1)

    mesh = plsc.VectorSubcoreMesh(core_axis_name="c", subcore_axis_name="s",
                                  num_cores=2, num_subcores=16)
    maxg = A_gncp.shape[1]
    maxm = A_outb.shape[1]
    maxc = A_csd.shape[1]
    out_flat = pl.kernel(
        _sc_body,
        out_type=jax.ShapeDtypeStruct((_K * _OUT[0] * 128 * _C,), jnp.float32),
        mesh=mesh,
        compiler_params=pltpu.CompilerParams(needs_layout_passes=False),
        scratch_types=[
            pltpu.VMEM((WINW,), jnp.float32),
            pltpu.VMEM((ACCW,), jnp.float32),
            pltpu.VMEM((maxn,), jnp.float32),
            pltpu.VMEM((maxn,), jnp.int32),
            pltpu.VMEM((128,), jnp.float32),
            pltpu.VMEM((16,), jnp.int32),
            pltpu.VMEM((maxg,), jnp.int32),
            pltpu.VMEM((maxg,), jnp.int32),
            pltpu.VMEM((maxc,), jnp.int32),
            pltpu.VMEM((maxc,), jnp.int32),
            pltpu.VMEM((maxm,), jnp.int32),
            pltpu.VMEM((maxm,), jnp.int32),
            pltpu.VMEM((maxn,), jnp.int32),
            pltpu.SemaphoreType.DMA,
        ],
    )(xrev, psi_flat, qw,
      jnp.asarray(A_widx), jnp.asarray(A_qwi),
      jnp.asarray(A_hdr), jnp.asarray(A_gncp), jnp.asarray(A_gnit),
      jnp.asarray(A_csd), jnp.asarray(A_css),
      jnp.asarray(A_outb), jnp.asarray(A_nzc4), jnp.asarray(A_s1))

    out = out_flat.reshape(_K, _OUT[0], 128, _C)
    out = jnp.transpose(out, (3, 0, 1, 2))
    return out.reshape(1, _C, _K, _OUT[0], _OUT[1])
